# Initial kernel scaffold; baseline (speedup 1.0000x reference)
import jax
import jax.numpy as jnp
from jax.experimental import pallas as pl


def kernel(preds, atoms, heads, bodies):
    raise NotImplementedError("write your pallas kernel here")


# speedup vs baseline: 3.2329x; 3.2329x over previous
import jax
import jax.numpy as jnp
from jax.experimental import pallas as pl
from jax.experimental.pallas import tpu as pltpu


def _constraints_kernel(
    heads_ref, bodies_hbm, x_hbm, out_ref, stage_ref, b_smem, sem, bsem
):
    tbl = out_ref.shape[-1]
    j = pl.program_id(0)
    cp = pltpu.make_async_copy(
        x_hbm.at[:, :, pl.ds(j * tbl, tbl)], out_ref, sem
    )
    cp.start()
    S, H = heads_ref.shape
    bcp = pltpu.make_async_copy(bodies_hbm.at[0], b_smem, bsem)
    bcp.start()
    cp.wait()
    for s in range(S):
        pltpu.make_async_copy(bodies_hbm.at[s], b_smem, bsem).wait()

        def compute(h, _, s=s):
            m = out_ref[b_smem[h, 0]]
            for k in range(1, bodies_hbm.shape[-1]):
                m = jnp.minimum(m, out_ref[b_smem[h, k]])
            stage_ref[h] = jnp.maximum(out_ref[heads_ref[s, h]], m)
            return 0

        jax.lax.fori_loop(0, H, compute, 0)

        if s + 1 < S:
            pltpu.make_async_copy(bodies_hbm.at[s + 1], b_smem, bsem).start()

        def apply(h, _, s=s):
            out_ref[heads_ref[s, h]] = stage_ref[h]
            return 0

        jax.lax.fori_loop(0, H, apply, 0)


def kernel(preds, atoms, heads, bodies, *, _tbl=128, _interpret=False):
    B, C = preds.shape
    S, H, K = bodies.shape
    sub = 8
    lanes = B // sub
    x = preds.T.reshape(C, sub, lanes)
    tbl = min(_tbl, lanes)
    grid = (lanes // tbl,)
    out = pl.pallas_call(
        _constraints_kernel,
        grid=grid,
        in_specs=[
            pl.BlockSpec(memory_space=pltpu.SMEM),
            pl.BlockSpec(memory_space=pl.ANY),
            pl.BlockSpec(memory_space=pl.ANY),
        ],
        out_specs=pl.BlockSpec((C, sub, tbl), lambda j: (0, 0, j)),
        out_shape=jax.ShapeDtypeStruct((C, sub, lanes), preds.dtype),
        scratch_shapes=[
            pltpu.VMEM((H, sub, tbl), preds.dtype),
            pltpu.SMEM((H, K), jnp.int32),
            pltpu.SemaphoreType.DMA,
            pltpu.SemaphoreType.DMA,
        ],
        compiler_params=pltpu.CompilerParams(
            dimension_semantics=("parallel",)
        ),
        interpret=_interpret,
    )(heads, bodies, x)
    return out.reshape(C, B).T


# speedup vs baseline: 5.0357x; 1.5576x over previous
import jax
import jax.numpy as jnp
from jax.experimental import pallas as pl
from jax.experimental.pallas import tpu as pltpu


def _constraints_kernel(
    heads_ref,
    bodies_hbm,
    x_hbm,
    out_hbm,
    work_ref,
    stage_ref,
    b_smem,
    sem,
    osem,
    bsem,
):
    tbl = work_ref.shape[-1]
    j = pl.program_id(0)
    cp = pltpu.make_async_copy(
        x_hbm.at[:, :, pl.ds(j * tbl, tbl)], work_ref, sem
    )
    cp.start()
    S, H = heads_ref.shape
    K = bodies_hbm.shape[-1]
    pltpu.make_async_copy(bodies_hbm.at[0], b_smem, bsem).start()
    cp.wait()
    for s in range(S):
        pltpu.make_async_copy(bodies_hbm.at[s], b_smem, bsem).wait()

        def compute(h, _, s=s):
            m = work_ref[b_smem[h, 0]]
            for k in range(1, K):
                m = jnp.minimum(m, work_ref[b_smem[h, k]])
            stage_ref[h] = jnp.maximum(work_ref[heads_ref[s, h]], m)
            return 0

        jax.lax.fori_loop(0, H, compute, 0, unroll=8)

        if s + 1 < S:
            pltpu.make_async_copy(bodies_hbm.at[s + 1], b_smem, bsem).start()

        def apply(h, _, s=s):
            work_ref[heads_ref[s, h]] = stage_ref[h]
            return 0

        jax.lax.fori_loop(0, H, apply, 0, unroll=8)

    ocp = pltpu.make_async_copy(
        work_ref, out_hbm.at[:, :, pl.ds(j * tbl, tbl)], osem
    )
    ocp.start()
    ocp.wait()


def kernel(preds, atoms, heads, bodies, *, _tbl=256, _interpret=False):
    B, C = preds.shape
    S, H, K = bodies.shape
    sub = 8
    lanes = B // sub
    x = preds.T.reshape(C, sub, lanes)
    tbl = min(_tbl, lanes)
    grid = (lanes // tbl,)
    out = pl.pallas_call(
        _constraints_kernel,
        grid=grid,
        in_specs=[
            pl.BlockSpec(memory_space=pltpu.SMEM),
            pl.BlockSpec(memory_space=pl.ANY),
            pl.BlockSpec(memory_space=pl.ANY),
        ],
        out_specs=pl.BlockSpec(memory_space=pl.ANY),
        out_shape=jax.ShapeDtypeStruct((C, sub, lanes), preds.dtype),
        scratch_shapes=[
            pltpu.VMEM((C, sub, tbl), preds.dtype),
            pltpu.VMEM((H, sub, tbl), preds.dtype),
            pltpu.SMEM((H, K), jnp.int32),
            pltpu.SemaphoreType.DMA,
            pltpu.SemaphoreType.DMA,
            pltpu.SemaphoreType.DMA,
        ],
        compiler_params=pltpu.CompilerParams(
            dimension_semantics=("parallel",)
        ),
        interpret=_interpret,
    )(heads, bodies, x)
    return out.reshape(C, B).T


# speedup vs baseline: 5.2023x; 1.0331x over previous
import jax
import jax.numpy as jnp
from jax.experimental import pallas as pl
from jax.experimental.pallas import tpu as pltpu


def _constraints_kernel(
    heads_ref,
    bodies_hbm,
    x_hbm,
    out_hbm,
    work_ref,
    stage_ref,
    b_smem,
    sem,
    osem,
    bsem,
):
    j = pl.program_id(0)
    cp = pltpu.make_async_copy(x_hbm.at[j], work_ref, sem)
    cp.start()
    S, H = heads_ref.shape
    K = bodies_hbm.shape[-1]
    pltpu.make_async_copy(bodies_hbm.at[0], b_smem, bsem).start()
    cp.wait()
    for s in range(S):
        pltpu.make_async_copy(bodies_hbm.at[s], b_smem, bsem).wait()

        def compute(h, _, s=s):
            m = work_ref[b_smem[h, 0]]
            for k in range(1, K):
                m = jnp.minimum(m, work_ref[b_smem[h, k]])
            stage_ref[h] = jnp.maximum(work_ref[heads_ref[s, h]], m)
            return 0

        jax.lax.fori_loop(0, H, compute, 0, unroll=8)

        if s + 1 < S:
            pltpu.make_async_copy(bodies_hbm.at[s + 1], b_smem, bsem).start()

        def apply(h, _, s=s):
            work_ref[heads_ref[s, h]] = stage_ref[h]
            return 0

        jax.lax.fori_loop(0, H, apply, 0, unroll=8)

    ocp = pltpu.make_async_copy(work_ref, out_hbm.at[j], osem)
    ocp.start()
    ocp.wait()


def kernel(preds, atoms, heads, bodies, *, _tbl=256, _interpret=False):
    B, C = preds.shape
    S, H, K = bodies.shape
    sub = 8
    lanes = B // sub
    tbl = min(_tbl, lanes)
    nsteps = lanes // tbl
    x = (
        preds.T.reshape(C, sub, nsteps, tbl)
        .transpose(2, 0, 1, 3)
    )
    grid = (nsteps,)
    out = pl.pallas_call(
        _constraints_kernel,
        grid=grid,
        in_specs=[
            pl.BlockSpec(memory_space=pltpu.SMEM),
            pl.BlockSpec(memory_space=pl.ANY),
            pl.BlockSpec(memory_space=pl.ANY),
        ],
        out_specs=pl.BlockSpec(memory_space=pl.ANY),
        out_shape=jax.ShapeDtypeStruct((nsteps, C, sub, tbl), preds.dtype),
        scratch_shapes=[
            pltpu.VMEM((C, sub, tbl), preds.dtype),
            pltpu.VMEM((H, sub, tbl), preds.dtype),
            pltpu.SMEM((H, K), jnp.int32),
            pltpu.SemaphoreType.DMA,
            pltpu.SemaphoreType.DMA,
            pltpu.SemaphoreType.DMA,
        ],
        compiler_params=pltpu.CompilerParams(
            dimension_semantics=("parallel",)
        ),
        interpret=_interpret,
    )(heads, bodies, x)
    return out.transpose(1, 2, 0, 3).reshape(C, B).T


# speedup vs baseline: 5.2348x; 1.0062x over previous
import jax
import jax.numpy as jnp
from jax.experimental import pallas as pl
from jax.experimental.pallas import tpu as pltpu


def _constraints_kernel(
    heads_ref,
    bodies_hbm,
    x_hbm,
    out_hbm,
    work_ref,
    stage_ref,
    b_smem,
    sem,
    osem,
    bsem,
):
    j = pl.program_id(0)
    cp = pltpu.make_async_copy(x_hbm.at[j], work_ref, sem)
    cp.start()
    S, H = heads_ref.shape
    K = bodies_hbm.shape[-1]
    pltpu.make_async_copy(bodies_hbm.at[0], b_smem, bsem).start()
    cp.wait()
    for s in range(S):
        pltpu.make_async_copy(bodies_hbm.at[s], b_smem, bsem).wait()

        def compute(h, _, s=s):
            m = work_ref[b_smem[h, 0]]
            for k in range(1, K):
                m = jnp.minimum(m, work_ref[b_smem[h, k]])
            stage_ref[h] = jnp.maximum(work_ref[heads_ref[s, h]], m)
            return 0

        jax.lax.fori_loop(0, H, compute, 0, unroll=16)

        if s + 1 < S:
            pltpu.make_async_copy(bodies_hbm.at[s + 1], b_smem, bsem).start()

        def apply(h, _, s=s):
            work_ref[heads_ref[s, h]] = stage_ref[h]
            return 0

        jax.lax.fori_loop(0, H, apply, 0, unroll=16)

    ocp = pltpu.make_async_copy(work_ref, out_hbm.at[j], osem)
    ocp.start()
    ocp.wait()


def kernel(preds, atoms, heads, bodies, *, _tbl=256, _interpret=False):
    B, C = preds.shape
    S, H, K = bodies.shape
    sub = 8
    lanes = B // sub
    tbl = min(_tbl, lanes)
    nsteps = lanes // tbl
    x = (
        preds.T.reshape(C, sub, nsteps, tbl)
        .transpose(2, 0, 1, 3)
    )
    grid = (nsteps,)
    out = pl.pallas_call(
        _constraints_kernel,
        grid=grid,
        in_specs=[
            pl.BlockSpec(memory_space=pltpu.SMEM),
            pl.BlockSpec(memory_space=pl.ANY),
            pl.BlockSpec(memory_space=pl.ANY),
        ],
        out_specs=pl.BlockSpec(memory_space=pl.ANY),
        out_shape=jax.ShapeDtypeStruct((nsteps, C, sub, tbl), preds.dtype),
        scratch_shapes=[
            pltpu.VMEM((C, sub, tbl), preds.dtype),
            pltpu.VMEM((H, sub, tbl), preds.dtype),
            pltpu.SMEM((H, K), jnp.int32),
            pltpu.SemaphoreType.DMA,
            pltpu.SemaphoreType.DMA,
            pltpu.SemaphoreType.DMA,
        ],
        compiler_params=pltpu.CompilerParams(
            dimension_semantics=("parallel",)
        ),
        interpret=_interpret,
    )(heads, bodies, x)
    return out.transpose(1, 2, 0, 3).reshape(C, B).T
